# baseline (speedup 1.0000x reference)
import functools

import jax
import jax.numpy as jnp
from jax import lax
from jax.experimental import pallas as pl
from jax.experimental.pallas import tpu as pltpu
from jax.experimental.pallas import tpu_sc as plsc
from jax.experimental.scheduling_groups import xla_metadata_call

R = 4096
K = 8192
D = 256
L = 16
NC, NS = 2, 16
NW = NC * NS
SC_N = 2048
SPLIT = R - SC_N
SC_ROWS_PER_W = SC_N // NW
G_ROWS_PER_W = SPLIT // NW
CHUNK_ROWS = 4
NCHUNKS = SC_ROWS_PER_W // CHUNK_ROWS
UNROLL = 8
WIN = UNROLL * L
NITER = K // WIN
BR = 512
NBLK = SPLIT // BR

_mesh = plsc.VectorSubcoreMesh(core_axis_name="c", subcore_axis_name="s")


def _merge(mx_a, it_a, mx_b, it_b):
    take_b = (mx_b > mx_a) | ((mx_b == mx_a) & (it_b < it_a))
    return jnp.where(take_b, mx_b, mx_a), jnp.where(take_b, it_b, it_a)


@functools.partial(
    pl.kernel,
    out_type=jax.ShapeDtypeStruct((SC_N, D), jnp.float32),
    mesh=_mesh,
    scratch_types=[
        pltpu.VMEM((CHUNK_ROWS, K), jnp.float32),
        pltpu.VMEM((CHUNK_ROWS, K), jnp.float32),
        pltpu.VMEM((SC_ROWS_PER_W,), jnp.int32),
        pltpu.VMEM((SC_ROWS_PER_W, D), jnp.float32),
        pltpu.SemaphoreType.DMA,
        pltpu.SemaphoreType.DMA,
        pltpu.SemaphoreType.DMA,
    ],
    compiler_params=pltpu.CompilerParams(needs_layout_passes=False),
)
def _quantize(w_hbm, cb_hbm, out_hbm, buf0, buf1, idx_v, rows_v, sem0, sem1,
              semg):
    wid = lax.axis_index("s") * NC + lax.axis_index("c")
    base = wid * SC_ROWS_PER_W
    obase = wid * SC_ROWS_PER_W
    lane = lax.broadcasted_iota(jnp.int32, (L,), 0)
    bufs = (buf0, buf1)
    sems = (sem0, sem1)

    def start(c, b):
        pltpu.make_async_copy(
            w_hbm.at[pl.ds(base + c * CHUNK_ROWS, CHUNK_ROWS)],
            bufs[b], sems[b]).start()

    def wait(b):
        pltpu.make_async_copy(
            w_hbm.at[pl.ds(base, CHUNK_ROWS)], bufs[b], sems[b]).wait()

    def process(buf, c):
        for r in range(CHUNK_ROWS):

            def step(j, carry):
                jv = jnp.full((L,), j, jnp.int32)
                out = list(carry)
                for u in range(UNROLL):
                    a = u // 2
                    mx, it = out[2 * a], out[2 * a + 1]
                    v = buf[r, pl.ds(j * WIN + u * L, L)]
                    m = v > mx
                    out[2 * a] = jnp.where(m, v, mx)
                    out[2 * a + 1] = jnp.where(m, jv, it)
                return tuple(out)

            init = []
            for _ in range(4):
                init += [jnp.full((L,), -jnp.inf, jnp.float32),
                         jnp.zeros((L,), jnp.int32)]
            acc = lax.fori_loop(0, NITER, step, tuple(init))

            mx01, it01 = _merge(acc[0], acc[1], acc[2], acc[3])
            mx23, it23 = _merge(acc[4], acc[5], acc[6], acc[7])
            vmax, vit = _merge(mx01, it01, mx23, it23)

            vbase = vit * WIN + lane
            rvec = jnp.full((L,), r, jnp.int32)
            fmin = jnp.full((L,), K, jnp.int32)
            for u in range(UNROLL):
                fidx = vbase + u * L
                val = plsc.load_gather(buf, [rvec, fidx])
                fmin = jnp.minimum(fmin, jnp.where(val == vmax, fidx, K))

            gmax = jnp.max(vmax)
            cand = jnp.where(vmax == gmax, fmin, jnp.int32(K))
            gidx = jnp.full((L,), jnp.min(cand), jnp.int32)
            pos = jnp.full((L,), c * CHUNK_ROWS + r, jnp.int32)
            plsc.store_scatter(idx_v, [pos], gidx, mask=lane == 0)

    start(0, 0)

    def pair_body(g, _):
        for b in range(2):
            c = g * 2 + b
            nxt = c + 1

            @pl.when(nxt < NCHUNKS)
            def _():
                start(nxt, 1 - b)

            wait(b)
            process(bufs[b], c)
        return 0

    lax.fori_loop(0, NCHUNKS // 2, pair_body, 0)
    pltpu.async_copy(cb_hbm.at[idx_v], rows_v, semg).wait()
    pltpu.sync_copy(rows_v, out_hbm.at[pl.ds(obase, SC_ROWS_PER_W)])


def _tc_body(w_ref, idx_ref):
    x = w_ref[...]
    m = jnp.max(x, axis=1, keepdims=True)
    ii = lax.broadcasted_iota(jnp.int32, x.shape, 1)
    cand = jnp.where(x == m, ii, jnp.int32(K))
    idx_ref[0, 0, :] = jnp.min(cand, axis=1)


_tc_argmax = pl.pallas_call(
    _tc_body,
    grid=(NBLK,),
    in_specs=[pl.BlockSpec((BR, K), lambda i: (i + SC_N // BR, 0))],
    out_specs=pl.BlockSpec((1, 1, BR), lambda i: (i, 0, 0)),
    out_shape=jax.ShapeDtypeStruct((NBLK, 1, BR), jnp.int32),
)


@functools.partial(
    pl.kernel,
    out_type=jax.ShapeDtypeStruct((SPLIT, D), jnp.float32),
    mesh=_mesh,
    scratch_types=[
        pltpu.VMEM((G_ROWS_PER_W,), jnp.int32),
        pltpu.VMEM((G_ROWS_PER_W, D), jnp.float32),
        pltpu.SemaphoreType.DMA,
    ],
    compiler_params=pltpu.CompilerParams(needs_layout_passes=False),
)
def _sc_gather(idx_hbm, cb_hbm, out_hbm, idx_v, rows_v, sem):
    wid = lax.axis_index("s") * NC + lax.axis_index("c")
    base = wid * G_ROWS_PER_W
    pltpu.sync_copy(idx_hbm.at[pl.ds(base, G_ROWS_PER_W)], idx_v)
    pltpu.async_copy(cb_hbm.at[idx_v], rows_v, sem).wait()
    pltpu.sync_copy(rows_v, out_hbm.at[pl.ds(base, G_ROWS_PER_W)])


def kernel(weights, codebook):
    w2 = weights.reshape(R, K)
    group = {"_scheduling_group_id": "0"}
    out_sc = xla_metadata_call(lambda a, b: _quantize(a, b), **group)(
        w2, codebook)
    idx_tc = xla_metadata_call(lambda a: _tc_argmax(a), **group)(
        w2).reshape(SPLIT)
    out_tc = _sc_gather(idx_tc, codebook)
    out = jnp.concatenate([out_sc, out_tc], axis=0)
    return out.reshape(weights.shape[0], weights.shape[1], D)


# speedup vs baseline: 1.3420x; 1.3420x over previous
import functools

import jax
import jax.numpy as jnp
from jax import lax
from jax.experimental import pallas as pl
from jax.experimental.pallas import tpu as pltpu
from jax.experimental.pallas import tpu_sc as plsc

R = 4096
K = 8192
D = 256
L = 16
NC, NS = 2, 16
NW = NC * NS
ROWS_PER_W = R // NW
BR = 512
NBLK = R // BR

_mesh = plsc.VectorSubcoreMesh(core_axis_name="c", subcore_axis_name="s")


def _tc_body(w_ref, idx_ref):
    x = w_ref[...]
    m = jnp.max(x, axis=1, keepdims=True)
    ii = lax.broadcasted_iota(jnp.int32, x.shape, 1)
    cand = jnp.where(x == m, ii, jnp.int32(K))
    idx_ref[0, 0, :] = jnp.min(cand, axis=1)


_tc_argmax = pl.pallas_call(
    _tc_body,
    grid=(NBLK,),
    in_specs=[pl.BlockSpec((BR, K), lambda i: (i, 0))],
    out_specs=pl.BlockSpec((1, 1, BR), lambda i: (i, 0, 0)),
    out_shape=jax.ShapeDtypeStruct((NBLK, 1, BR), jnp.int32),
)


@functools.partial(
    pl.kernel,
    out_type=jax.ShapeDtypeStruct((R, D), jnp.float32),
    mesh=_mesh,
    scratch_types=[
        pltpu.VMEM((ROWS_PER_W,), jnp.int32),
        pltpu.VMEM((ROWS_PER_W, D), jnp.float32),
        pltpu.SemaphoreType.DMA,
        pltpu.SemaphoreType.DMA,
        pltpu.SemaphoreType.DMA,
    ],
    compiler_params=pltpu.CompilerParams(needs_layout_passes=False),
)
def _sc_gather(idx_hbm, cb_hbm, out_hbm, idx_v, rows_v, sem, semb, semw):
    wid = lax.axis_index("s") * NC + lax.axis_index("c")
    base = wid * ROWS_PER_W
    half = ROWS_PER_W // 2
    pltpu.sync_copy(idx_hbm.at[pl.ds(base, ROWS_PER_W)], idx_v)
    g0 = pltpu.make_async_copy(
        cb_hbm.at[idx_v.at[pl.ds(0, half)]], rows_v.at[pl.ds(0, half)], sem)
    g1 = pltpu.make_async_copy(
        cb_hbm.at[idx_v.at[pl.ds(half, half)]],
        rows_v.at[pl.ds(half, half)], semb)
    g0.start()
    g1.start()
    g0.wait()
    w0 = pltpu.make_async_copy(
        rows_v.at[pl.ds(0, half)], out_hbm.at[pl.ds(base, half)], semw)
    w0.start()
    g1.wait()
    pltpu.sync_copy(
        rows_v.at[pl.ds(half, half)], out_hbm.at[pl.ds(base + half, half)])
    w0.wait()


def kernel(weights, codebook):
    idx = _tc_argmax(weights.reshape(R, K)).reshape(R)
    out = _sc_gather(idx, codebook)
    return out.reshape(weights.shape[0], weights.shape[1], D)
